# baseline (speedup 1.0000x reference)
import functools
import math

import jax
import jax.numpy as jnp
from jax import lax
from jax.experimental import pallas as pl
from jax.experimental.pallas import tpu as pltpu
from jax.experimental.pallas import tpu_sc as plsc

SEQ = 16
B = 8
D_IN = 512
D_QKV = 512
N = 512
R = SEQ * B

NC = 2
NS = 16
NW = NC * NS
R_SC = 0
R_TC = R - R_SC
G = 8
CN = 64
NCH = N // CN
NDC = D_IN // 16
WPR = NW // max(R_SC, 1)
CPW = NCH // max(WPR, 1)


def _qt_body(xq_ref, wq_ref, wk_ref, qt_ref):
    q = lax.dot_general(
        xq_ref[...], wq_ref[...],
        dimension_numbers=(((1,), (1,)), ((), ())),
        preferred_element_type=jnp.float32,
    )
    qt_ref[...] = lax.dot_general(
        q, wk_ref[...],
        dimension_numbers=(((1,), (0,)), ((), ())),
        preferred_element_type=jnp.float32,
    ) * (1.0 / math.sqrt(D_QKV))


def _compute_qt(xq, Wq, Wk):
    return pl.pallas_call(
        _qt_body,
        out_shape=jax.ShapeDtypeStruct((R, D_IN), jnp.float32),
    )(xq, Wq, Wk)


def _permute(x, idx):
    dn = lax.GatherDimensionNumbers(
        offset_dims=(), collapsed_slice_dims=(0,), start_index_map=(0,))
    return lax.gather(x, idx[:, None], dn, slice_sizes=(1,),
                      mode=lax.GatherScatterMode.PROMISE_IN_BOUNDS)


_sc_mesh = plsc.VectorSubcoreMesh(core_axis_name="c", subcore_axis_name="s")


@functools.partial(
    pl.kernel,
    out_type=jax.ShapeDtypeStruct((R_SC, N), jnp.float32),
    mesh=_sc_mesh,
    scratch_types=[
        pltpu.VMEM((2, CN, D_IN), jnp.float32),
        pltpu.VMEM((D_IN,), jnp.float32),
        pltpu.VMEM((CPW * CN,), jnp.float32),
        pltpu.SemaphoreType.DMA,
        pltpu.SemaphoreType.DMA,
    ],
)
def _sc_dot(kv_hbm, qt_hbm, out_hbm, kv_buf, qt_buf, out_buf, sem_a, sem_b):
    wid = lax.axis_index("s") * NC + lax.axis_index("c")
    out_row = wid // WPR
    row = R_TC + out_row
    cstart = (wid % WPR) * CPW

    lane = lax.iota(jnp.int32, 16)

    pltpu.sync_copy(qt_hbm.at[row], qt_buf)
    qs = [qt_buf[pl.ds(dc * 16, 16)] for dc in range(NDC)]

    def compute_chunk(bsel, lbase):
        def group_body(g, _):
            vec = jnp.zeros((16,), jnp.float32)
            for t in range(16):
                n = g * 16 + t
                accs = [None] * 4
                for a in range(4):
                    acc = kv_buf[bsel, n, pl.ds(a * 128, 16)] * qs[a * 8]
                    for j in range(1, 8):
                        dc = a * 8 + j
                        acc = acc + kv_buf[bsel, n, pl.ds(dc * 16, 16)] * qs[dc]
                    accs[a] = acc
                total = (accs[0] + accs[1]) + (accs[2] + accs[3])
                for k in (1, 2, 4, 8):
                    total = total + _permute(total, lane ^ k)
                vec = jnp.where(lane == t, total, vec)
            out_buf[pl.ds(lbase + g * 16, 16)] = vec
            return 0

        lax.fori_loop(0, CN // 16, group_body, 0)

    pltpu.async_copy(
        kv_hbm.at[row, pl.ds(cstart * CN, CN), :], kv_buf.at[0], sem_a)

    def chunk2_body(c2, carry2):
        c0 = cstart + c2 * 2
        pltpu.async_copy(
            kv_hbm.at[row, pl.ds((c0 + 1) * CN, CN), :], kv_buf.at[1], sem_b)
        pltpu.make_async_copy(
            kv_hbm.at[row, pl.ds(c0 * CN, CN), :], kv_buf.at[0], sem_a).wait()
        compute_chunk(0, c2 * 2 * CN)

        @pl.when(c2 < CPW // 2 - 1)
        def _prefetch_even():
            pltpu.async_copy(
                kv_hbm.at[row, pl.ds((c0 + 2) * CN, CN), :], kv_buf.at[0],
                sem_a)

        pltpu.make_async_copy(
            kv_hbm.at[row, pl.ds((c0 + 1) * CN, CN), :], kv_buf.at[1],
            sem_b).wait()
        compute_chunk(1, (c2 * 2 + 1) * CN)
        return 0

    lax.fori_loop(0, CPW // 2, chunk2_body, 0)
    pltpu.sync_copy(
        out_buf, out_hbm.at[out_row, pl.ds(cstart * CN, CPW * CN)])


def _tc_dot_body(qt_ref, kv_ref, out_ref):
    kvb = kv_ref[...].reshape(G, N, D_IN)
    out_ref[...] = jnp.sum(kvb * qt_ref[...][:, None, :], axis=-1)


def _tc_dot(kv2, qt):
    return pl.pallas_call(
        _tc_dot_body,
        grid=(R_TC // G,),
        in_specs=[
            pl.BlockSpec((G, D_IN), lambda i: (i, 0)),
            pl.BlockSpec((G * N, D_IN), lambda i: (i, 0)),
        ],
        out_specs=pl.BlockSpec((G, N), lambda i: (i, 0)),
        out_shape=jax.ShapeDtypeStruct((R_TC, N), jnp.float32),
    )(qt, kv2)


@jax.jit
def _run(xq, kv, Wq, Wk):
    qt = _compute_qt(xq, Wq, Wk)
    if R_SC == 0:
        return _tc_dot(kv.reshape(R * N, D_IN), qt)
    qk_sc = _sc_dot(kv, qt)
    qk_tc = _tc_dot(kv, qt)
    return jnp.concatenate([qk_tc, qk_sc], axis=0)


def kernel(input_q, input_kv, Wq, Wk):
    xq = input_q.reshape(R, D_IN)
    kv = input_kv.reshape(R, N, D_IN)
    qk = _run(xq, kv, Wq, Wk)
    return qk.reshape(SEQ, B, N)


# speedup vs baseline: 1.0224x; 1.0224x over previous
import functools
import math

import jax
import jax.numpy as jnp
from jax import lax
from jax.experimental import pallas as pl
from jax.experimental.pallas import tpu as pltpu
from jax.experimental.pallas import tpu_sc as plsc

SEQ = 16
B = 8
D_IN = 512
D_QKV = 512
N = 512
R = SEQ * B

NC = 2
NS = 16
NW = NC * NS
R_SC = 0
R_TC = R - R_SC
G = 8
CN = 64
NCH = N // CN
NDC = D_IN // 16
WPR = NW // max(R_SC, 1)
CPW = NCH // max(WPR, 1)


def _qt_body(xq_ref, wq_ref, wk_ref, qt_ref):
    q = lax.dot_general(
        xq_ref[...], wq_ref[...],
        dimension_numbers=(((1,), (1,)), ((), ())),
        preferred_element_type=jnp.float32,
    )
    qt_ref[...] = lax.dot_general(
        q, wk_ref[...],
        dimension_numbers=(((1,), (0,)), ((), ())),
        preferred_element_type=jnp.float32,
    ) * (1.0 / math.sqrt(D_QKV))


def _compute_qt(xq, Wq, Wk):
    return pl.pallas_call(
        _qt_body,
        out_shape=jax.ShapeDtypeStruct((R, D_IN), jnp.float32),
    )(xq, Wq, Wk)


def _permute(x, idx):
    dn = lax.GatherDimensionNumbers(
        offset_dims=(), collapsed_slice_dims=(0,), start_index_map=(0,))
    return lax.gather(x, idx[:, None], dn, slice_sizes=(1,),
                      mode=lax.GatherScatterMode.PROMISE_IN_BOUNDS)


_sc_mesh = plsc.VectorSubcoreMesh(core_axis_name="c", subcore_axis_name="s")


@functools.partial(
    pl.kernel,
    out_type=jax.ShapeDtypeStruct((R_SC, N), jnp.float32),
    mesh=_sc_mesh,
    scratch_types=[
        pltpu.VMEM((2, CN, D_IN), jnp.float32),
        pltpu.VMEM((D_IN,), jnp.float32),
        pltpu.VMEM((CPW * CN,), jnp.float32),
        pltpu.SemaphoreType.DMA,
        pltpu.SemaphoreType.DMA,
    ],
)
def _sc_dot(kv_hbm, qt_hbm, out_hbm, kv_buf, qt_buf, out_buf, sem_a, sem_b):
    wid = lax.axis_index("s") * NC + lax.axis_index("c")
    out_row = wid // WPR
    row = R_TC + out_row
    cstart = (wid % WPR) * CPW

    lane = lax.iota(jnp.int32, 16)

    pltpu.sync_copy(qt_hbm.at[row], qt_buf)
    qs = [qt_buf[pl.ds(dc * 16, 16)] for dc in range(NDC)]

    def compute_chunk(bsel, lbase):
        def group_body(g, _):
            vec = jnp.zeros((16,), jnp.float32)
            for t in range(16):
                n = g * 16 + t
                accs = [None] * 4
                for a in range(4):
                    acc = kv_buf[bsel, n, pl.ds(a * 128, 16)] * qs[a * 8]
                    for j in range(1, 8):
                        dc = a * 8 + j
                        acc = acc + kv_buf[bsel, n, pl.ds(dc * 16, 16)] * qs[dc]
                    accs[a] = acc
                total = (accs[0] + accs[1]) + (accs[2] + accs[3])
                for k in (1, 2, 4, 8):
                    total = total + _permute(total, lane ^ k)
                vec = jnp.where(lane == t, total, vec)
            out_buf[pl.ds(lbase + g * 16, 16)] = vec
            return 0

        lax.fori_loop(0, CN // 16, group_body, 0)

    pltpu.async_copy(
        kv_hbm.at[row, pl.ds(cstart * CN, CN), :], kv_buf.at[0], sem_a)

    def chunk2_body(c2, carry2):
        c0 = cstart + c2 * 2
        pltpu.async_copy(
            kv_hbm.at[row, pl.ds((c0 + 1) * CN, CN), :], kv_buf.at[1], sem_b)
        pltpu.make_async_copy(
            kv_hbm.at[row, pl.ds(c0 * CN, CN), :], kv_buf.at[0], sem_a).wait()
        compute_chunk(0, c2 * 2 * CN)

        @pl.when(c2 < CPW // 2 - 1)
        def _prefetch_even():
            pltpu.async_copy(
                kv_hbm.at[row, pl.ds((c0 + 2) * CN, CN), :], kv_buf.at[0],
                sem_a)

        pltpu.make_async_copy(
            kv_hbm.at[row, pl.ds((c0 + 1) * CN, CN), :], kv_buf.at[1],
            sem_b).wait()
        compute_chunk(1, (c2 * 2 + 1) * CN)
        return 0

    lax.fori_loop(0, CPW // 2, chunk2_body, 0)
    pltpu.sync_copy(
        out_buf, out_hbm.at[out_row, pl.ds(cstart * CN, CPW * CN)])


def _tc_dot_body(xq_ref, wq_ref, wk_ref, kva_ref, kvb_ref, out_ref):
    q = lax.dot_general(
        xq_ref[...], wq_ref[...],
        dimension_numbers=(((1,), (1,)), ((), ())),
        preferred_element_type=jnp.float32,
    )
    qt = lax.dot_general(
        q, wk_ref[...],
        dimension_numbers=(((1,), (0,)), ((), ())),
        preferred_element_type=jnp.float32,
    ) * (1.0 / math.sqrt(D_QKV))
    h = G // 2
    out_ref[0:h, :] = jnp.sum(kva_ref[...] * qt[0:h, None, :], axis=-1)
    out_ref[h:G, :] = jnp.sum(kvb_ref[...] * qt[h:G, None, :], axis=-1)


def _tc_dot(kv, xq, Wq, Wk):
    h = G // 2
    return pl.pallas_call(
        _tc_dot_body,
        grid=(R_TC // G,),
        in_specs=[
            pl.BlockSpec((G, D_IN), lambda i: (i, 0)),
            pl.BlockSpec((D_QKV, D_IN), lambda i: (0, 0)),
            pl.BlockSpec((D_QKV, D_IN), lambda i: (0, 0)),
            pl.BlockSpec((h, N, D_IN), lambda i: (2 * i, 0, 0)),
            pl.BlockSpec((h, N, D_IN), lambda i: (2 * i + 1, 0, 0)),
        ],
        out_specs=pl.BlockSpec((G, N), lambda i: (i, 0)),
        out_shape=jax.ShapeDtypeStruct((R_TC, N), jnp.float32),
    )(xq, Wq, Wk, kv, kv)


@jax.jit
def _run(xq, kv, Wq, Wk):
    if R_SC == 0:
        return _tc_dot(kv, xq, Wq, Wk)
    qt = _compute_qt(xq, Wq, Wk)
    qk_sc = _sc_dot(kv, qt)
    qk_tc = _tc_dot(kv, xq, Wq, Wk)
    return jnp.concatenate([qk_tc, qk_sc], axis=0)


def kernel(input_q, input_kv, Wq, Wk):
    xq = input_q.reshape(R, D_IN)
    kv = input_kv.reshape(R, N, D_IN)
    qk = _run(xq, kv, Wq, Wk)
    return qk.reshape(SEQ, B, N)


# speedup vs baseline: 1.0523x; 1.0293x over previous
import math

import jax
import jax.numpy as jnp
from jax import lax
from jax.experimental import pallas as pl

SEQ = 16
B = 8
D_IN = 512
D_QKV = 512
N = 512
R = SEQ * B
G = 8


def _body(xq_ref, wq_ref, wk_ref, kv_ref, out_ref):
    q = lax.dot_general(
        xq_ref[...], wq_ref[...],
        dimension_numbers=(((1,), (1,)), ((), ())),
        preferred_element_type=jnp.float32,
    )
    qt = lax.dot_general(
        q, wk_ref[...],
        dimension_numbers=(((1,), (0,)), ((), ())),
        preferred_element_type=jnp.float32,
    ) * (1.0 / math.sqrt(D_QKV))
    out_ref[...] = jnp.sum(kv_ref[...] * qt[:, None, :], axis=-1)


@jax.jit
def _run(xq, kv, Wq, Wk):
    return pl.pallas_call(
        _body,
        grid=(R // G,),
        in_specs=[
            pl.BlockSpec((G, D_IN), lambda i: (i, 0)),
            pl.BlockSpec((D_QKV, D_IN), lambda i: (0, 0)),
            pl.BlockSpec((D_QKV, D_IN), lambda i: (0, 0)),
            pl.BlockSpec((G, N, D_IN), lambda i: (i, 0, 0)),
        ],
        out_specs=pl.BlockSpec((G, N), lambda i: (i, 0)),
        out_shape=jax.ShapeDtypeStruct((R, N), jnp.float32),
    )(xq, Wq, Wk, kv)


def kernel(input_q, input_kv, Wq, Wk):
    xq = input_q.reshape(R, D_IN)
    kv = input_kv.reshape(R, N, D_IN)
    qk = _run(xq, kv, Wq, Wk)
    return qk.reshape(SEQ, B, N)
